# baseline (speedup 1.0000x reference)
import functools

import jax
import jax.numpy as jnp
from jax import lax
from jax.experimental import pallas as pl
from jax.experimental.pallas import tpu as pltpu
from jax.experimental.pallas import tpu_sc as plsc

B, N, C, G = 16, 20000, 80, 32
CH = 1280
K = -(-N // CH)


def _iou_argmax_body(bpT_ref, gt_ref, gt0_ref, idx_ref, bxp_ref,
                     mx_ref, ai_ref, bx_ref):
    b = pl.program_id(0)
    k = pl.program_id(1)

    @pl.when(k == 0)
    def _():
        mx_ref[...] = jnp.full((G, 1), -jnp.inf, jnp.float32)
        ai_ref[...] = jnp.zeros((G, 1), jnp.int32)
        bx_ref[...] = jnp.zeros((G, 4), jnp.float32)

    bp = bpT_ref[0]
    x1p = bp[0:1, :]
    y1p = bp[1:2, :]
    x2p = bp[2:3, :]
    y2p = bp[3:4, :]
    area_p = (x2p - x1p) * (y2p - y1p)

    gt = gt_ref[0]
    gx1 = gt[:, 0:1]
    gy1 = gt[:, 1:2]
    gx2 = gt[:, 2:3]
    gy2 = gt[:, 3:4]
    area_g = (gx2 - gx1) * (gy2 - gy1)

    w = jnp.maximum(jnp.minimum(gx2, x2p) - jnp.maximum(gx1, x1p), 0.0)
    h = jnp.maximum(jnp.minimum(gy2, y2p) - jnp.maximum(gy1, y1p), 0.0)
    inter = w * h
    union = (area_g + area_p) - inter
    lane = lax.broadcasted_iota(jnp.int32, (G, CH), 1)
    gidx = lane + k * CH
    iou = jnp.where(gidx < N, inter / union, -jnp.inf)

    m = jnp.max(iou, axis=1, keepdims=True)
    aidx = jnp.min(jnp.where(iou == m, gidx, N), axis=1, keepdims=True)
    one = gidx == aidx
    bx1 = jnp.sum(jnp.where(one, x1p, 0.0), axis=1, keepdims=True)
    by1 = jnp.sum(jnp.where(one, y1p, 0.0), axis=1, keepdims=True)
    bx2 = jnp.sum(jnp.where(one, x2p, 0.0), axis=1, keepdims=True)
    by2 = jnp.sum(jnp.where(one, y2p, 0.0), axis=1, keepdims=True)
    nbox = jnp.concatenate([bx1, by1, bx2, by2], axis=1)

    upd = m > mx_ref[...]
    mx_ref[...] = jnp.where(upd, m, mx_ref[...])
    ai_ref[...] = jnp.where(upd, aidx, ai_ref[...])
    bx_ref[...] = jnp.where(upd, nbox, bx_ref[...])

    idx_ref[0] = ai_ref[...] + b * N

    tgt = gt0_ref[0, pl.ds(b, 1), :]
    d = bx_ref[...] - tgt
    ad = jnp.abs(d)
    sl1 = jnp.where(ad < 1.0, 0.5 * d * d, ad - 0.5)
    bxp_ref[...] = jnp.reshape(jnp.sum(sl1), (1, 1, 1))


def _loss_body(x_ref, lbl_ref, bxp_ref, out_ref):
    x = x_ref[...]
    lbl = lbl_ref[...]
    iota = lax.broadcasted_iota(jnp.int32, (B * G, C), 1)
    z = (iota == jnp.clip(lbl, 0, C - 1)).astype(jnp.float32)
    bce = jnp.maximum(x, 0.0) - x * z + jnp.log(1.0 + jnp.exp(-jnp.abs(x)))
    total = jnp.sum(bce) / (B * G * C) + jnp.sum(bxp_ref[...]) / (B * G * 4)
    out_ref[...] = jnp.reshape(total, (1, 1))


_NC, _NS = 2, 16
_NW = _NC * _NS
_RPW = (B * G) // _NW


@functools.cache
def _make_sc_gather():
    @functools.partial(
        pl.kernel,
        out_type=jax.ShapeDtypeStruct((B * G, C), jnp.float32),
        mesh=plsc.VectorSubcoreMesh(core_axis_name="c", subcore_axis_name="s"),
        scratch_types=[
            pltpu.VMEM((_RPW,), jnp.int32),
            pltpu.VMEM((_RPW, C), jnp.float32),
            pltpu.SemaphoreType.DMA,
        ],
        compiler_params=pltpu.CompilerParams(use_tc_tiling_on_sc=False),
    )
    def _sc_gather(table_hbm, idx_hbm, out_hbm, idx_v, rows_v, sem):
        wid = lax.axis_index("s") * _NC + lax.axis_index("c")
        base = wid * _RPW
        pltpu.sync_copy(idx_hbm.at[pl.ds(base, _RPW)], idx_v)
        pltpu.async_copy(table_hbm.at[idx_v], rows_v, sem).wait()
        pltpu.sync_copy(rows_v, out_hbm.at[pl.ds(base, _RPW)])

    return _sc_gather


def _stage_a(bpT, gt_boxes, interpret=False):
    return pl.pallas_call(
        _iou_argmax_body,
        grid=(B, K),
        in_specs=[
            pl.BlockSpec((1, 4, CH), lambda b, k: (b, 0, k)),
            pl.BlockSpec((1, G, 4), lambda b, k: (b, 0, 0)),
            pl.BlockSpec((1, G, 4), lambda b, k: (0, 0, 0)),
        ],
        out_specs=[
            pl.BlockSpec((1, G, 1), lambda b, k: (b, 0, 0)),
            pl.BlockSpec((1, 1, 1), lambda b, k: (b, 0, 0)),
        ],
        out_shape=[
            jax.ShapeDtypeStruct((B, G, 1), jnp.int32),
            jax.ShapeDtypeStruct((B, 1, 1), jnp.float32),
        ],
        scratch_shapes=[
            pltpu.VMEM((G, 1), jnp.float32),
            pltpu.VMEM((G, 1), jnp.int32),
            pltpu.VMEM((G, 4), jnp.float32),
        ],
        compiler_params=pltpu.CompilerParams(
            dimension_semantics=("arbitrary", "arbitrary")),
        interpret=interpret,
    )(bpT, gt_boxes, gt_boxes)


def _stage_c(gathered, lbl2, bxp, interpret=False):
    return pl.pallas_call(
        _loss_body,
        out_shape=jax.ShapeDtypeStruct((1, 1), jnp.float32),
        interpret=interpret,
    )(gathered, lbl2, bxp)


def kernel(cls_preds, box_preds, gt_boxes, gt_labels):
    bpT = jnp.transpose(box_preds, (0, 2, 1))
    flat_idx3, bxp = _stage_a(bpT, gt_boxes)
    flat_idx = flat_idx3.reshape(B * G)
    gathered = _make_sc_gather()(cls_preds.reshape(B * N, C), flat_idx)
    lbl2 = gt_labels.reshape(B * G, 1)
    out = _stage_c(gathered, lbl2, bxp)
    return out.reshape(())
